# baseline (speedup 1.0000x reference)
import functools

import numpy as np

import jax
import jax.numpy as jnp
from jax import lax
from jax.experimental import pallas as pl
from jax.experimental.pallas import tpu as pltpu
from jax.experimental.pallas import tpu_sc as plsc

N = 10000
E = 320000
D = 128
L = 3

NC = 2
NS = 16
NP = 10240
WPT = NP // NS
EPSC = E // NC
EPT = EPSC // NS
B = 40
NCH = EPT // B

_PI = np.concatenate(
    [np.concatenate([32 * k + 2 * np.arange(16),
                     32 * k + 2 * np.arange(16) + 1]) for k in range(4)])
_INV = np.argsort(_PI)


def _sc_agg(h_pk, ea_pk, src, dst):
    mesh = plsc.VectorSubcoreMesh(core_axis_name="c", subcore_axis_name="s")

    @functools.partial(
        pl.kernel,
        out_type=jax.ShapeDtypeStruct((NC, NP, D), jnp.float32),
        mesh=mesh,
        compiler_params=pltpu.CompilerParams(needs_layout_passes=False),
        scratch_types=[
            pltpu.VMEM_SHARED((NP, D), jnp.float32),
            pltpu.VMEM((B,), jnp.int32),
            pltpu.VMEM((B,), jnp.int32),
            pltpu.VMEM((2, B), jnp.int32),
            pltpu.VMEM((2, B), jnp.int32),
            pltpu.VMEM((B, D), jnp.float32),
            pltpu.VMEM((B, D), jnp.float32),
            pltpu.VMEM((B, D // 2), jnp.int32),
            pltpu.VMEM((B, D // 2), jnp.int32),
            pltpu.VMEM((B, D), jnp.float32),
            pltpu.VMEM((B, D), jnp.float32),
            pltpu.SemaphoreType.DMA,
            pltpu.SemaphoreType.DMA,
            pltpu.SemaphoreType.DMA,
            pltpu.SemaphoreType.DMA,
            pltpu.SemaphoreType.DMA,
            pltpu.SemaphoreType.DMA,
            pltpu.SemaphoreType.DMA,
            pltpu.SemaphoreType.DMA,
            pltpu.SemaphoreType.DMA,
            pltpu.SemaphoreType.DMA,
            pltpu.SemaphoreType.DMA,
            pltpu.SemaphoreType.DMA,
            pltpu.SemaphoreType.DMA,
            pltpu.SemaphoreType.DMA,
        ],
    )
    def k(h_hbm, ea_hbm, src_hbm, dst_hbm, out_hbm, agg_spm,
          srcix0, srcix1, dstix0, dstix1, rows0, rows1, eav0, eav1,
          msg0, msg1, isem0, isem1, esem0, esem1, gsem0, gsem1,
          dsem00, dsem01, dsem10, dsem11, ssem00, ssem01, ssem10, ssem11):
        c = lax.axis_index("c")
        s = lax.axis_index("s")
        base = (c * NS + s) * EPT

        srcix = (srcix0, srcix1)
        dstix = (dstix0, dstix1)
        rows = (rows0, rows1)
        eav = (eav0, eav1)
        msg = (msg0, msg1)
        isem = (isem0, isem1)
        esem = (esem0, esem1)
        gsem = (gsem0, gsem1)
        dsem = ((dsem00, dsem01), (dsem10, dsem11))
        ssem = ((ssem00, ssem01), (ssem10, ssem11))

        def issue_srcix(kc, u):
            e0 = pl.multiple_of(base + kc * B, 8)
            pltpu.async_copy(src_hbm.at[pl.ds(e0, B)], srcix[u], isem[u])

        def wait_srcix(u):
            pltpu.make_async_copy(src_hbm.at[pl.ds(0, B)], srcix[u],
                                  isem[u]).wait()

        def issue_dstix(kc, u, p):
            e0 = pl.multiple_of(base + kc * B, 8)
            pltpu.async_copy(dst_hbm.at[pl.ds(e0, B)], dstix[u].at[p],
                             dsem[u][p])

        def wait_dstix(u, p):
            pltpu.make_async_copy(dst_hbm.at[pl.ds(0, B)], dstix[u].at[p],
                                  dsem[u][p]).wait()

        def issue_ea(kc, u):
            r0 = pl.multiple_of(base + kc * B, 8)
            pltpu.async_copy(ea_hbm.at[pl.ds(r0, B)], eav[u], esem[u])

        def wait_ea(u):
            pltpu.make_async_copy(ea_hbm.at[pl.ds(0, B)], eav[u],
                                  esem[u]).wait()

        def issue_gather(u):
            pltpu.async_copy(h_hbm.at[srcix[u]], rows[u], gsem[u])

        def wait_gather(u):
            pltpu.make_async_copy(h_hbm.at[pl.ds(0, B)], rows[u],
                                  gsem[u]).wait()

        def issue_scatter(u, p):
            pltpu.async_copy(msg[u], agg_spm.at[dstix[u].at[p]],
                             ssem[u][p], add=True)

        def wait_scatter(u, p):
            pltpu.make_async_copy(msg[u], agg_spm.at[dstix[u].at[p]],
                                  ssem[u][p]).wait()

        def compute(u):
            r_ref, e_ref, m_ref = rows[u], eav[u], msg[u]

            @plsc.parallel_loop(0, B, unroll=2)
            def _(ei):
                for jj in range(4):
                    ew = e_ref[ei, pl.ds(16 * jj, 16)]
                    lo, hi = plsc.unpack(
                        plsc.bitcast(ew, jnp.bfloat16),
                        format=plsc.PackFormat.INTERLEAVED)
                    ha = r_ref[ei, pl.ds(32 * jj, 16)]
                    hb = r_ref[ei, pl.ds(32 * jj + 16, 16)]
                    m_ref[ei, pl.ds(32 * jj, 16)] = jnp.maximum(
                        ha + lo, 0.0)
                    m_ref[ei, pl.ds(32 * jj + 16, 16)] = jnp.maximum(
                        hb + hi, 0.0)

        def half(x, u, p, first, xs):
            wait_gather(u)

            @pl.when(xs + 2 < NCH)
            def _():
                issue_srcix(xs + 2, u)

            wait_ea(u)
            if not first:
                wait_scatter(u, 1 - p)

            @pl.when(xs + 2 < NCH)
            def _():
                issue_dstix(xs + 2, u, 1 - p)

            compute(u)

            @pl.when(xs + 2 < NCH)
            def _():
                issue_ea(xs + 2, u)

            wait_dstix(u, p)
            issue_scatter(u, p)

            @pl.when(xs + 2 < NCH)
            def _():
                wait_srcix(u)
                issue_gather(u)

        def zbody(i, carry):
            for j in range(D // 16):
                msg0[i, pl.ds(j * 16, 16)] = jnp.zeros((16,), jnp.float32)
            return carry
        lax.fori_loop(0, B, zbody, 0)

        def zcopy(q, carry):
            pltpu.sync_copy(msg0, agg_spm.at[pl.ds(s * WPT + q * B, B)])
            return carry
        lax.fori_loop(0, WPT // B, zcopy, 0)

        for u in (0, 1):
            issue_srcix(u, u)
            issue_dstix(u, u, 0)
            issue_ea(u, u)
        for u in (0, 1):
            wait_srcix(u)
            issue_gather(u)
        plsc.subcore_barrier()

        for x in range(4):
            half(x, x % 2, (x // 2) % 2, first=(x < 2), xs=x)

        def quad(t, carry):
            for q in range(4):
                half(None, q % 2, q // 2, first=False, xs=4 * t + q)
            return carry
        lax.fori_loop(1, NCH // 4, quad, 0)

        half(NCH - 2, 0, 0, first=False, xs=NCH - 2)
        half(NCH - 1, 1, 0, first=False, xs=NCH - 1)
        wait_scatter(0, 0)
        wait_scatter(1, 0)
        plsc.subcore_barrier()

        def wcopy(q, carry):
            r0 = s * WPT + q * B
            pltpu.sync_copy(agg_spm.at[pl.ds(r0, B)], msg0)
            pltpu.sync_copy(msg0, out_hbm.at[c, pl.ds(pl.multiple_of(r0, 8),
                                                      B)])
            return carry
        lax.fori_loop(0, WPT // B, wcopy, 0)

    return k(h_pk, ea_pk, src, dst)


def _tc_layer(h, agg0, agg1, Wl, bl, gl, betal, epsl, final):
    def body(h_ref, a0_ref, a1_ref, w_ref, b_ref, g_ref, bt_ref, e_ref,
             o_ref):
        t = (1.0 + e_ref[0, 0]) * h_ref[...] + (a0_ref[...] + a1_ref[...])
        z = jnp.dot(t, w_ref[...], preferred_element_type=jnp.float32)
        z = z + b_ref[...]
        m = jnp.mean(z, axis=0, keepdims=True)
        v = jnp.mean(jnp.square(z - m), axis=0, keepdims=True)
        zn = (z - m) * lax.rsqrt(v + 1e-5) * g_ref[...] + bt_ref[...]
        o = zn + h_ref[...]
        if final:
            o = jnp.maximum(o, 0.0)
        o_ref[...] = o

    return pl.pallas_call(
        body,
        out_shape=jax.ShapeDtypeStruct((N, D), jnp.float32),
    )(h, agg0, agg1, Wl, bl.reshape(1, D), gl.reshape(1, D),
      betal.reshape(1, D), epsl.reshape(1, 1))


def kernel(x, edge_index, edge_attr, eps, W, b, gamma, beta):
    src = edge_index[0]
    dst = edge_index[1]
    pi = jnp.asarray(_PI)
    inv = jnp.asarray(_INV)
    ea_bf = lax.bitcast_convert_type(
        edge_attr.astype(jnp.bfloat16).reshape(E, D // 2, 2), jnp.int32)
    h_nat = x
    h_pi = x[:, pi]
    for l in range(L):
        agg = _sc_agg(h_pi, ea_bf, src, dst)
        h_pi = _tc_layer(h_pi, agg[0, :N], agg[1, :N],
                         W[l][pi][:, pi], b[l][pi], gamma[l][pi],
                         beta[l][pi], eps[l], final=(l == L - 1))
    return h_pi[:, inv]


# speedup vs baseline: 1.0430x; 1.0430x over previous
import functools

import numpy as np

import jax
import jax.numpy as jnp
from jax import lax
from jax.experimental import pallas as pl
from jax.experimental.pallas import tpu as pltpu
from jax.experimental.pallas import tpu_sc as plsc

N = 10000
E = 320000
D = 128
L = 3

NC = 2
NS = 16
EPSC = E // NC
EPT = EPSC // NS
B = 80
NCH = EPT // B

_PI = np.concatenate(
    [np.concatenate([32 * k + 2 * np.arange(16),
                     32 * k + 2 * np.arange(16) + 1]) for k in range(4)])
_INV = np.argsort(_PI)


def _sc_agg(h_pk, ea_pk, src, dst):
    mesh = plsc.VectorSubcoreMesh(core_axis_name="c", subcore_axis_name="s")

    @functools.partial(
        pl.kernel,
        out_type=jax.ShapeDtypeStruct((NC, N, D), jnp.float32),
        mesh=mesh,
        compiler_params=pltpu.CompilerParams(needs_layout_passes=False),
        scratch_types=[
            pltpu.VMEM_SHARED((N, D), jnp.float32),
            pltpu.VMEM((B,), jnp.int32),
            pltpu.VMEM((B,), jnp.int32),
            pltpu.VMEM((1, B), jnp.int32),
            pltpu.VMEM((1, B), jnp.int32),
            pltpu.VMEM((B, D), jnp.float32),
            pltpu.VMEM((B, D), jnp.float32),
            pltpu.VMEM((B // 2, D), jnp.int32),
            pltpu.VMEM((B // 2, D), jnp.int32),
            pltpu.VMEM((B, D), jnp.float32),
            pltpu.SemaphoreType.DMA,
            pltpu.SemaphoreType.DMA,
            pltpu.SemaphoreType.DMA,
            pltpu.SemaphoreType.DMA,
            pltpu.SemaphoreType.DMA,
            pltpu.SemaphoreType.DMA,
            pltpu.SemaphoreType.DMA,
            pltpu.SemaphoreType.DMA,
            pltpu.SemaphoreType.DMA,
        ],
    )
    def k(h_hbm, ea_hbm, src_hbm, dst_hbm, out_hbm, agg_spm,
          srcix0, srcix1, dstix0, dstix1, rows0, rows1, eav0, eav1, msgb,
          isem0, isem1, dsem0, dsem1, esem0, esem1, gsem0, gsem1, ssem):
        c = lax.axis_index("c")
        s = lax.axis_index("s")
        base = (c * NS + s) * EPT

        srcix = (srcix0, srcix1)
        dstix = (dstix0, dstix1)
        rows = (rows0, rows1)
        eav = (eav0, eav1)
        isem = (isem0, isem1)
        dsem = (dsem0, dsem1)
        esem = (esem0, esem1)
        gsem = (gsem0, gsem1)

        def issue_srcix(kc, u):
            e0 = pl.multiple_of(base + kc * B, 8)
            pltpu.async_copy(src_hbm.at[pl.ds(e0, B)], srcix[u], isem[u])

        def wait_srcix(u):
            pltpu.make_async_copy(src_hbm.at[pl.ds(0, B)], srcix[u],
                                  isem[u]).wait()

        def issue_dstix(kc, u):
            e0 = pl.multiple_of(base + kc * B, 8)
            pltpu.async_copy(dst_hbm.at[pl.ds(e0, B)], dstix[u].at[0],
                             dsem[u])

        def wait_dstix(u):
            pltpu.make_async_copy(dst_hbm.at[pl.ds(0, B)], dstix[u].at[0],
                                  dsem[u]).wait()

        def issue_ea(kc, u):
            r0 = pl.multiple_of((base + kc * B) // 2, 8)
            pltpu.async_copy(ea_hbm.at[pl.ds(r0, B // 2)], eav[u], esem[u])

        def wait_ea(u):
            pltpu.make_async_copy(ea_hbm.at[pl.ds(0, B // 2)], eav[u],
                                  esem[u]).wait()

        def issue_gather(u):
            pltpu.async_copy(h_hbm.at[srcix[u]], rows[u], gsem[u])

        def wait_gather(u):
            pltpu.make_async_copy(h_hbm.at[pl.ds(0, B)], rows[u],
                                  gsem[u]).wait()

        def issue_scatter(u):
            pltpu.async_copy(msgb, agg_spm.at[dstix[u].at[0]], ssem,
                             add=True)

        def wait_scatter(u):
            pltpu.make_async_copy(msgb, agg_spm.at[dstix[u].at[0]],
                                  ssem).wait()

        def compute(u):
            r_ref, e_ref = rows[u], eav[u]

            @plsc.parallel_loop(0, B // 2, unroll=2)
            def _(ri):
                for je in range(2):
                    ei = 2 * ri + je
                    for jj in range(4):
                        ew = e_ref[ri, pl.ds(64 * je + 16 * jj, 16)]
                        lo, hi = plsc.unpack(
                            plsc.bitcast(ew, jnp.bfloat16),
                            format=plsc.PackFormat.INTERLEAVED)
                        ha = r_ref[ei, pl.ds(32 * jj, 16)]
                        hb = r_ref[ei, pl.ds(32 * jj + 16, 16)]
                        msgb[ei, pl.ds(32 * jj, 16)] = jnp.maximum(
                            ha + lo, 0.0)
                        msgb[ei, pl.ds(32 * jj + 16, 16)] = jnp.maximum(
                            hb + hi, 0.0)

        def half(u, first, xs):
            wait_gather(u)

            @pl.when(xs + 2 < NCH)
            def _():
                issue_srcix(xs + 2, u)

            wait_ea(u)
            if not first:
                wait_scatter(u)

                @pl.when(xs + 1 < NCH)
                def _():
                    issue_dstix(xs + 1, 1 - u)
            compute(u)

            @pl.when(xs + 2 < NCH)
            def _():
                issue_ea(xs + 2, u)

            wait_dstix(u)
            issue_scatter(u)

            @pl.when(xs + 2 < NCH)
            def _():
                wait_srcix(u)
                issue_gather(u)

        def zbody(i, carry):
            for j in range(D // 16):
                msgb[i, pl.ds(j * 16, 16)] = jnp.zeros((16,), jnp.float32)
            return carry
        lax.fori_loop(0, B, zbody, 0)

        def zcopy(q, carry):
            pltpu.sync_copy(msgb.at[pl.ds(0, 25)],
                            agg_spm.at[pl.ds(s * (N // NS) + q * 25, 25)])
            return carry
        lax.fori_loop(0, (N // NS) // 25, zcopy, 0)

        for u in (0, 1):
            issue_srcix(u, u)
            issue_dstix(u, u)
            issue_ea(u, u)
        for u in (0, 1):
            wait_srcix(u)
            issue_gather(u)
        plsc.subcore_barrier()

        half(0, True, 0)
        half(1, False, 1)

        def pair(t, carry):
            half(0, False, 2 * t)
            half(1, False, 2 * t + 1)
            return carry
        lax.fori_loop(1, NCH // 2, pair, 0)

        half(0, False, NCH - 1)
        wait_scatter(0)
        plsc.subcore_barrier()

        def wcopy(q, carry):
            r0 = s * 640 + q * B
            pltpu.sync_copy(agg_spm.at[pl.ds(r0, B)], msgb)
            pltpu.sync_copy(msgb, out_hbm.at[c, pl.ds(pl.multiple_of(r0, 8),
                                                      B)])
            return carry

        @pl.when(s < NS - 1)
        def _():
            lax.fori_loop(0, 8, wcopy, 0)

        @pl.when(s == NS - 1)
        def _():
            lax.fori_loop(0, 5, wcopy, 0)

    return k(h_pk, ea_pk, src, dst)


def _tc_layer(h, agg0, agg1, Wl, bl, gl, betal, epsl, final):
    def body(h_ref, a0_ref, a1_ref, w_ref, b_ref, g_ref, bt_ref, e_ref,
             o_ref):
        t = (1.0 + e_ref[0, 0]) * h_ref[...] + (a0_ref[...] + a1_ref[...])
        z = jnp.dot(t, w_ref[...], preferred_element_type=jnp.float32)
        z = z + b_ref[...]
        m = jnp.mean(z, axis=0, keepdims=True)
        v = jnp.mean(jnp.square(z - m), axis=0, keepdims=True)
        zn = (z - m) * lax.rsqrt(v + 1e-5) * g_ref[...] + bt_ref[...]
        o = zn + h_ref[...]
        if final:
            o = jnp.maximum(o, 0.0)
        o_ref[...] = o

    return pl.pallas_call(
        body,
        out_shape=jax.ShapeDtypeStruct((N, D), jnp.float32),
    )(h, agg0, agg1, Wl, bl.reshape(1, D), gl.reshape(1, D),
      betal.reshape(1, D), epsl.reshape(1, 1))


def kernel(x, edge_index, edge_attr, eps, W, b, gamma, beta):
    src = edge_index[0]
    dst = edge_index[1]
    pi = jnp.asarray(_PI)
    inv = jnp.asarray(_INV)
    ea_bf = lax.bitcast_convert_type(
        edge_attr.astype(jnp.bfloat16).reshape(E, D // 2, 2),
        jnp.int32).reshape(E // 2, D)
    h_pi = x[:, pi]
    for l in range(L):
        agg = _sc_agg(h_pi, ea_bf, src, dst)
        h_pi = _tc_layer(h_pi, agg[0], agg[1],
                         W[l][pi][:, pi], b[l][pi], gamma[l][pi],
                         beta[l][pi], eps[l], final=(l == L - 1))
    return h_pi[:, inv]


# speedup vs baseline: 1.7080x; 1.6377x over previous
import functools

import numpy as np

import jax
import jax.numpy as jnp
from jax import lax
from jax.experimental import pallas as pl
from jax.experimental.pallas import tpu as pltpu
from jax.experimental.pallas import tpu_sc as plsc

N = 10000
E = 320000
D = 128
L = 3

NC = 2
NS = 16
EPSC = E // NC
EPT = EPSC // NS
B = 80
NCH = EPT // B

def _sc_agg(h_pk, ea_pk, src, dst):
    mesh = plsc.VectorSubcoreMesh(core_axis_name="c", subcore_axis_name="s")

    @functools.partial(
        pl.kernel,
        out_type=jax.ShapeDtypeStruct((NC, N, D), jnp.float32),
        mesh=mesh,
        compiler_params=pltpu.CompilerParams(needs_layout_passes=False),
        scratch_types=[
            pltpu.VMEM_SHARED((N, D), jnp.float32),
            pltpu.VMEM((B,), jnp.int32),
            pltpu.VMEM((B,), jnp.int32),
            pltpu.VMEM((1, B), jnp.int32),
            pltpu.VMEM((1, B), jnp.int32),
            pltpu.VMEM((B, D), jnp.float32),
            pltpu.VMEM((B, D), jnp.float32),
            pltpu.VMEM((B, D // 2), jnp.int32),
            pltpu.VMEM((B, D), jnp.float32),
            pltpu.SemaphoreType.DMA,
            pltpu.SemaphoreType.DMA,
            pltpu.SemaphoreType.DMA,
            pltpu.SemaphoreType.DMA,
            pltpu.SemaphoreType.DMA,
            pltpu.SemaphoreType.DMA,
            pltpu.SemaphoreType.DMA,
            pltpu.SemaphoreType.DMA,
        ],
    )
    def k(h_hbm, ea_hbm, src_hbm, dst_hbm, out_hbm, agg_spm,
          srcix0, srcix1, dstix0, dstix1, rows0, rows1, eavb, msgb,
          isem0, isem1, dsem0, dsem1, esem, gsem0, gsem1, ssem):
        c = lax.axis_index("c")
        s = lax.axis_index("s")
        base = (c * NS + s) * EPT

        srcix = (srcix0, srcix1)
        dstix = (dstix0, dstix1)
        rows = (rows0, rows1)
        isem = (isem0, isem1)
        dsem = (dsem0, dsem1)
        gsem = (gsem0, gsem1)

        def issue_srcix(kc, u):
            e0 = pl.multiple_of(base + kc * B, 8)
            pltpu.async_copy(src_hbm.at[pl.ds(e0, B)], srcix[u], isem[u])

        def wait_srcix(u):
            pltpu.make_async_copy(src_hbm.at[pl.ds(0, B)], srcix[u],
                                  isem[u]).wait()

        def issue_dstix(kc, u):
            e0 = pl.multiple_of(base + kc * B, 8)
            pltpu.async_copy(dst_hbm.at[pl.ds(e0, B)], dstix[u].at[0],
                             dsem[u])

        def wait_dstix(u):
            pltpu.make_async_copy(dst_hbm.at[pl.ds(0, B)], dstix[u].at[0],
                                  dsem[u]).wait()

        def issue_ea(kc):
            r0 = pl.multiple_of(base + kc * B, 8)
            pltpu.async_copy(ea_hbm.at[pl.ds(r0, B)], eavb, esem)

        def wait_ea():
            pltpu.make_async_copy(ea_hbm.at[pl.ds(0, B)], eavb,
                                  esem).wait()

        def issue_gather(u):
            pltpu.async_copy(h_hbm.at[srcix[u]], rows[u], gsem[u])

        def wait_gather(u):
            pltpu.make_async_copy(h_hbm.at[pl.ds(0, B)], rows[u],
                                  gsem[u]).wait()

        def issue_scatter(u):
            pltpu.async_copy(msgb, agg_spm.at[dstix[u].at[0]], ssem,
                             add=True)

        def wait_scatter(u):
            pltpu.make_async_copy(msgb, agg_spm.at[dstix[u].at[0]],
                                  ssem).wait()

        def compute(u):
            r_ref, e_ref = rows[u], eavb

            @plsc.parallel_loop(0, B, unroll=2)
            def _(ei):
                for jj in range(4):
                    ew = e_ref[ei, pl.ds(16 * jj, 16)]
                    lo, hi = plsc.unpack(
                        plsc.bitcast(ew, jnp.bfloat16),
                        format=plsc.PackFormat.INTERLEAVED)
                    ha = r_ref[ei, pl.ds(16 * jj, 16)]
                    hb = r_ref[ei, pl.ds(64 + 16 * jj, 16)]
                    msgb[ei, pl.ds(16 * jj, 16)] = jnp.maximum(
                        ha + lo, 0.0)
                    msgb[ei, pl.ds(64 + 16 * jj, 16)] = jnp.maximum(
                        hb + hi, 0.0)

        def half(u, first, xs):
            wait_gather(u)

            @pl.when(xs + 2 < NCH)
            def _():
                issue_srcix(xs + 2, u)

            wait_ea()
            if not first:
                wait_scatter(u)

                @pl.when(xs + 1 < NCH)
                def _():
                    issue_dstix(xs + 1, 1 - u)
            compute(u)

            @pl.when(xs + 1 < NCH)
            def _():
                issue_ea(xs + 1)

            wait_dstix(u)
            issue_scatter(u)

            @pl.when(xs + 2 < NCH)
            def _():
                wait_srcix(u)
                issue_gather(u)

        def zbody(i, carry):
            for j in range(D // 16):
                msgb[i, pl.ds(j * 16, 16)] = jnp.zeros((16,), jnp.float32)
            return carry
        lax.fori_loop(0, B, zbody, 0)

        def zcopy(q, carry):
            pltpu.sync_copy(msgb.at[pl.ds(0, 25)],
                            agg_spm.at[pl.ds(s * (N // NS) + q * 25, 25)])
            return carry
        lax.fori_loop(0, (N // NS) // 25, zcopy, 0)

        for u in (0, 1):
            issue_srcix(u, u)
            issue_dstix(u, u)
        issue_ea(0)
        for u in (0, 1):
            wait_srcix(u)
            issue_gather(u)
        plsc.subcore_barrier()

        half(0, True, 0)
        half(1, False, 1)

        def pair(t, carry):
            half(0, False, 2 * t)
            half(1, False, 2 * t + 1)
            return carry
        lax.fori_loop(1, NCH // 2, pair, 0)

        half(0, False, NCH - 1)
        wait_scatter(0)
        plsc.subcore_barrier()

        def wcopy(q, carry):
            r0 = s * 640 + q * B
            pltpu.sync_copy(agg_spm.at[pl.ds(r0, B)], msgb)
            pltpu.sync_copy(msgb, out_hbm.at[c, pl.ds(pl.multiple_of(r0, 8),
                                                      B)])
            return carry

        @pl.when(s < NS - 1)
        def _():
            lax.fori_loop(0, 8, wcopy, 0)

        @pl.when(s == NS - 1)
        def _():
            lax.fori_loop(0, 5, wcopy, 0)

    return k(h_pk, ea_pk, src, dst)


def _tc_layer(h, agg, Wl, bl, gl, betal, epsl, final):
    def body(h_ref, a_ref, w_ref, b_ref, g_ref, bt_ref, e_ref, o_ref):
        t = (1.0 + e_ref[0, 0]) * h_ref[...] + (a_ref[0] + a_ref[1])
        z = jnp.dot(t, w_ref[...], preferred_element_type=jnp.float32)
        z = z + b_ref[...]
        m = jnp.mean(z, axis=0, keepdims=True)
        v = jnp.mean(jnp.square(z - m), axis=0, keepdims=True)
        zn = (z - m) * lax.rsqrt(v + 1e-5) * g_ref[...] + bt_ref[...]
        o = zn + h_ref[...]
        if final:
            o = jnp.maximum(o, 0.0)
        o_ref[...] = o

    return pl.pallas_call(
        body,
        out_shape=jax.ShapeDtypeStruct((N, D), jnp.float32),
    )(h, agg, Wl, bl.reshape(1, D), gl.reshape(1, D),
      betal.reshape(1, D), epsl.reshape(1, 1))


def _pack_ea(edge_attr):
    def body(x_ref, o_ref):
        xb = x_ref[...]
        lo = xb[:, 0:64].astype(jnp.bfloat16)
        hi = xb[:, 64:128].astype(jnp.bfloat16)
        lo32 = lax.bitcast_convert_type(lo, jnp.uint16).astype(jnp.uint32)
        hi32 = lax.bitcast_convert_type(hi, jnp.uint16).astype(jnp.uint32)
        o_ref[...] = lax.bitcast_convert_type(lo32 | (hi32 << 16),
                                              jnp.int32)

    RB = 20000
    return pl.pallas_call(
        body,
        grid=(E // RB,),
        in_specs=[pl.BlockSpec((RB, D), lambda i: (i, 0))],
        out_specs=pl.BlockSpec((RB, D // 2), lambda i: (i, 0)),
        out_shape=jax.ShapeDtypeStruct((E, D // 2), jnp.int32),
    )(edge_attr)


def kernel(x, edge_index, edge_attr, eps, W, b, gamma, beta):
    src = edge_index[0]
    dst = edge_index[1]
    ea_pk = _pack_ea(edge_attr)
    h = x
    for l in range(L):
        agg = _sc_agg(h, ea_pk, src, dst)
        h = _tc_layer(h, agg, W[l], b[l], gamma[l], beta[l], eps[l],
                      final=(l == L - 1))
    return h


# speedup vs baseline: 1.9299x; 1.1299x over previous
import functools

import numpy as np

import jax
import jax.numpy as jnp
from jax import lax
from jax.experimental import pallas as pl
from jax.experimental.pallas import tpu as pltpu
from jax.experimental.pallas import tpu_sc as plsc

N = 10000
E = 320000
D = 128
L = 3

NC = 2
NS = 16
EPSC = E // NC
EPT = EPSC // NS
B = 80
NCH = EPT // B

def _sc_agg(h_pk, ea_pk, src, dst):
    mesh = plsc.VectorSubcoreMesh(core_axis_name="c", subcore_axis_name="s")

    @functools.partial(
        pl.kernel,
        out_type=jax.ShapeDtypeStruct((NC, N, D), jnp.float32),
        mesh=mesh,
        compiler_params=pltpu.CompilerParams(needs_layout_passes=False),
        scratch_types=[
            pltpu.VMEM_SHARED((N, D), jnp.float32),
            pltpu.VMEM((B,), jnp.int32),
            pltpu.VMEM((B,), jnp.int32),
            pltpu.VMEM((1, B), jnp.int32),
            pltpu.VMEM((1, B), jnp.int32),
            pltpu.VMEM((B, D), jnp.float32),
            pltpu.VMEM((B, D), jnp.float32),
            pltpu.VMEM((B // 2, D), jnp.int32),
            pltpu.VMEM((B // 2, D), jnp.int32),
            pltpu.VMEM((B, D), jnp.float32),
            pltpu.SemaphoreType.DMA,
            pltpu.SemaphoreType.DMA,
            pltpu.SemaphoreType.DMA,
            pltpu.SemaphoreType.DMA,
            pltpu.SemaphoreType.DMA,
            pltpu.SemaphoreType.DMA,
            pltpu.SemaphoreType.DMA,
            pltpu.SemaphoreType.DMA,
            pltpu.SemaphoreType.DMA,
        ],
    )
    def k(h_hbm, ea_hbm, src_hbm, dst_hbm, out_hbm, agg_spm,
          srcix0, srcix1, dstix0, dstix1, rows0, rows1, eav0, eav1, msgb,
          isem0, isem1, dsem0, dsem1, esem0, esem1, gsem0, gsem1, ssem):
        c = lax.axis_index("c")
        s = lax.axis_index("s")
        base = (c * NS + s) * EPT

        srcix = (srcix0, srcix1)
        dstix = (dstix0, dstix1)
        rows = (rows0, rows1)
        eav = (eav0, eav1)
        isem = (isem0, isem1)
        dsem = (dsem0, dsem1)
        esem = (esem0, esem1)
        gsem = (gsem0, gsem1)

        def issue_srcix(kc, u):
            e0 = pl.multiple_of(base + kc * B, 8)
            pltpu.async_copy(src_hbm.at[pl.ds(e0, B)], srcix[u], isem[u])

        def wait_srcix(u):
            pltpu.make_async_copy(src_hbm.at[pl.ds(0, B)], srcix[u],
                                  isem[u]).wait()

        def issue_dstix(kc, u):
            e0 = pl.multiple_of(base + kc * B, 8)
            pltpu.async_copy(dst_hbm.at[pl.ds(e0, B)], dstix[u].at[0],
                             dsem[u])

        def wait_dstix(u):
            pltpu.make_async_copy(dst_hbm.at[pl.ds(0, B)], dstix[u].at[0],
                                  dsem[u]).wait()

        def issue_ea(kc, u):
            r0 = pl.multiple_of((base + kc * B) // 2, 8)
            pltpu.async_copy(ea_hbm.at[pl.ds(r0, B // 2)], eav[u], esem[u])

        def wait_ea(u):
            pltpu.make_async_copy(ea_hbm.at[pl.ds(0, B // 2)], eav[u],
                                  esem[u]).wait()

        def issue_gather(u):
            pltpu.async_copy(h_hbm.at[srcix[u]], rows[u], gsem[u])

        def wait_gather(u):
            pltpu.make_async_copy(h_hbm.at[pl.ds(0, B)], rows[u],
                                  gsem[u]).wait()

        def issue_scatter(u):
            pltpu.async_copy(msgb, agg_spm.at[dstix[u].at[0]], ssem,
                             add=True)

        def wait_scatter(u):
            pltpu.make_async_copy(msgb, agg_spm.at[dstix[u].at[0]],
                                  ssem).wait()

        def compute(u):
            r_ref, e_ref = rows[u], eav[u]

            @plsc.parallel_loop(0, B // 2, unroll=2)
            def _(ri):
                for je in range(2):
                    ei = ri + 40 * je
                    for jj in range(4):
                        ew = e_ref[ri, pl.ds(64 * je + 16 * jj, 16)]
                        lo, hi = plsc.unpack(
                            plsc.bitcast(ew, jnp.bfloat16),
                            format=plsc.PackFormat.INTERLEAVED)
                        ha = r_ref[ei, pl.ds(16 * jj, 16)]
                        hb = r_ref[ei, pl.ds(64 + 16 * jj, 16)]
                        msgb[ei, pl.ds(16 * jj, 16)] = jnp.maximum(
                            ha + lo, 0.0)
                        msgb[ei, pl.ds(64 + 16 * jj, 16)] = jnp.maximum(
                            hb + hi, 0.0)

        def half(u, first, xs):
            wait_gather(u)

            @pl.when(xs + 2 < NCH)
            def _():
                issue_srcix(xs + 2, u)

            wait_ea(u)
            if not first:
                wait_scatter(u)

                @pl.when(xs + 1 < NCH)
                def _():
                    issue_dstix(xs + 1, 1 - u)
            compute(u)

            @pl.when(xs + 2 < NCH)
            def _():
                issue_ea(xs + 2, u)

            wait_dstix(u)
            issue_scatter(u)

            @pl.when(xs + 2 < NCH)
            def _():
                wait_srcix(u)
                issue_gather(u)

        def zbody(i, carry):
            for j in range(D // 16):
                msgb[i, pl.ds(j * 16, 16)] = jnp.zeros((16,), jnp.float32)
            return carry
        lax.fori_loop(0, B, zbody, 0)

        def zcopy(q, carry):
            pltpu.sync_copy(msgb.at[pl.ds(0, 25)],
                            agg_spm.at[pl.ds(s * (N // NS) + q * 25, 25)])
            return carry
        lax.fori_loop(0, (N // NS) // 25, zcopy, 0)

        for u in (0, 1):
            issue_srcix(u, u)
            issue_dstix(u, u)
            issue_ea(u, u)
        for u in (0, 1):
            wait_srcix(u)
            issue_gather(u)
        plsc.subcore_barrier()

        half(0, True, 0)
        half(1, False, 1)

        def pair(t, carry):
            half(0, False, 2 * t)
            half(1, False, 2 * t + 1)
            return carry
        lax.fori_loop(1, NCH // 2, pair, 0)

        half(0, False, NCH - 1)
        wait_scatter(0)
        plsc.subcore_barrier()

        def wcopy(q, carry):
            r0 = s * 640 + q * B
            pltpu.sync_copy(agg_spm.at[pl.ds(r0, B)], msgb)
            pltpu.sync_copy(msgb, out_hbm.at[c, pl.ds(pl.multiple_of(r0, 8),
                                                      B)])
            return carry

        @pl.when(s < NS - 1)
        def _():
            lax.fori_loop(0, 8, wcopy, 0)

        @pl.when(s == NS - 1)
        def _():
            lax.fori_loop(0, 5, wcopy, 0)

    return k(h_pk, ea_pk, src, dst)


def _tc_layer(h, agg, Wl, bl, gl, betal, epsl, final):
    def body(h_ref, a_ref, w_ref, b_ref, g_ref, bt_ref, e_ref, o_ref):
        t = (1.0 + e_ref[0, 0]) * h_ref[...] + (a_ref[0] + a_ref[1])
        z = jnp.dot(t, w_ref[...], preferred_element_type=jnp.float32)
        z = z + b_ref[...]
        m = jnp.mean(z, axis=0, keepdims=True)
        v = jnp.mean(jnp.square(z - m), axis=0, keepdims=True)
        zn = (z - m) * lax.rsqrt(v + 1e-5) * g_ref[...] + bt_ref[...]
        o = zn + h_ref[...]
        if final:
            o = jnp.maximum(o, 0.0)
        o_ref[...] = o

    return pl.pallas_call(
        body,
        out_shape=jax.ShapeDtypeStruct((N, D), jnp.float32),
    )(h, agg, Wl, bl.reshape(1, D), gl.reshape(1, D),
      betal.reshape(1, D), epsl.reshape(1, 1))


def _pack_ea(edge_attr):
    def pack_half(xb):
        lo = xb[:, 0:64].astype(jnp.bfloat16)
        hi = xb[:, 64:128].astype(jnp.bfloat16)
        lo32 = lax.bitcast_convert_type(lo, jnp.uint16).astype(jnp.uint32)
        hi32 = lax.bitcast_convert_type(hi, jnp.uint16).astype(jnp.uint32)
        return lax.bitcast_convert_type(lo32 | (hi32 << 16), jnp.int32)

    RB = 8000
    def body(x_ref, o_ref):
        for sc_i in range(RB // B):
            a = pack_half(x_ref[pl.ds(B * sc_i, B // 2)])
            bb = pack_half(x_ref[pl.ds(B * sc_i + B // 2, B // 2)])
            o_ref[pl.ds((B // 2) * sc_i, B // 2), 0:64] = a
            o_ref[pl.ds((B // 2) * sc_i, B // 2), 64:128] = bb

    return pl.pallas_call(
        body,
        grid=(E // RB,),
        in_specs=[pl.BlockSpec((RB, D), lambda i: (i, 0))],
        out_specs=pl.BlockSpec((RB // 2, D), lambda i: (i, 0)),
        out_shape=jax.ShapeDtypeStruct((E // 2, D), jnp.int32),
    )(edge_attr)


def kernel(x, edge_index, edge_attr, eps, W, b, gamma, beta):
    src = edge_index[0]
    dst = edge_index[1]
    ea_pk = _pack_ea(edge_attr)
    h = x
    for l in range(L):
        agg = _sc_agg(h, ea_pk, src, dst)
        h = _tc_layer(h, agg, W[l], b[l], gamma[l], beta[l], eps[l],
                      final=(l == L - 1))
    return h


# speedup vs baseline: 2.3769x; 1.2316x over previous
import functools

import jax
import jax.numpy as jnp
from jax import lax
from jax.experimental import pallas as pl
from jax.experimental.pallas import tpu as pltpu
from jax.experimental.pallas import tpu_sc as plsc

N = 10000
E = 320000
D = 128
L = 3

NC = 2
NS = 16
NP = 10240
WPT = NP // NS
EPSC = E // NC
EPT = EPSC // NS
B = 80
NCH = EPT // B


def _sc_agg(h, edge_attr, src, dst):
    mesh = plsc.VectorSubcoreMesh(core_axis_name="c", subcore_axis_name="s")

    @functools.partial(
        pl.kernel,
        out_type=jax.ShapeDtypeStruct((NC, NP, D), jnp.float32),
        mesh=mesh,
        scratch_types=[
            pltpu.VMEM_SHARED((NP, D), jnp.float32),
            pltpu.VMEM((B,), jnp.int32),
            pltpu.VMEM((B,), jnp.int32),
            pltpu.VMEM((1, B), jnp.int32),
            pltpu.VMEM((1, B), jnp.int32),
            pltpu.VMEM((B, D), jnp.float32),
            pltpu.VMEM((B, D), jnp.float32),
            pltpu.VMEM((B, D), jnp.float32),
            pltpu.VMEM((B, D), jnp.float32),
            pltpu.SemaphoreType.DMA,
            pltpu.SemaphoreType.DMA,
            pltpu.SemaphoreType.DMA,
            pltpu.SemaphoreType.DMA,
            pltpu.SemaphoreType.DMA,
            pltpu.SemaphoreType.DMA,
            pltpu.SemaphoreType.DMA,
            pltpu.SemaphoreType.DMA,
        ],
    )
    def k(h_hbm, ea_hbm, src_hbm, dst_hbm, out_hbm, agg_spm,
          srcix0, srcix1, dstix0, dstix1, rows0, rows1, eav0, eav1,
          isem0, isem1, dsem0, dsem1, esem0, esem1, gsem0, gsem1):
        c = lax.axis_index("c")
        s = lax.axis_index("s")
        base = (c * NS + s) * EPT

        slots = ((srcix0, dstix0, rows0, eav0, isem0, dsem0, esem0, gsem0),
                 (srcix1, dstix1, rows1, eav1, isem1, dsem1, esem1, gsem1))

        def e_off(kc):
            return pl.multiple_of(base + kc * B, 8)

        def issue_srcix(kc, u):
            pltpu.async_copy(src_hbm.at[pl.ds(e_off(kc), B)], slots[u][0],
                             slots[u][4])

        def wait_srcix(u):
            pltpu.make_async_copy(src_hbm.at[pl.ds(0, B)], slots[u][0],
                                  slots[u][4]).wait()

        def issue_dstix(kc, u):
            pltpu.async_copy(dst_hbm.at[pl.ds(e_off(kc), B)],
                             slots[u][1].at[0], slots[u][5])

        def wait_dstix(u):
            pltpu.make_async_copy(dst_hbm.at[pl.ds(0, B)],
                                  slots[u][1].at[0], slots[u][5]).wait()

        def issue_ea(kc, u):
            pltpu.async_copy(ea_hbm.at[pl.ds(e_off(kc), B)], slots[u][3],
                             slots[u][6])

        def wait_ea(u):
            pltpu.make_async_copy(ea_hbm.at[pl.ds(0, B)], slots[u][3],
                                  slots[u][6]).wait()

        def issue_gather(u):
            pltpu.async_copy(h_hbm.at[slots[u][0]], slots[u][2],
                             slots[u][7])

        def wait_gather(u):
            pltpu.make_async_copy(ea_hbm.at[pl.ds(0, B)], slots[u][2],
                                  slots[u][7]).wait()

        def compute(u):
            rows, eav = slots[u][2], slots[u][3]

            @plsc.parallel_loop(0, B, unroll=2)
            def _(bi):
                for j in range(D // 16):
                    sl = pl.ds(j * 16, 16)
                    eav[bi, sl] = jnp.maximum(rows[bi, sl] + eav[bi, sl],
                                              0.0)

        def scatter(u):
            pltpu.sync_copy(slots[u][3], agg_spm.at[slots[u][1].at[0]],
                            add=True)

        def zbody(i, carry):
            for j in range(D // 16):
                rows0[i, pl.ds(j * 16, 16)] = jnp.zeros((16,), jnp.float32)
            return carry
        lax.fori_loop(0, B, zbody, 0)

        def zcopy(q, carry):
            pltpu.sync_copy(rows0, agg_spm.at[pl.ds(s * WPT + q * B, B)])
            return carry
        lax.fori_loop(0, WPT // B, zcopy, 0)

        for u in (0, 1):
            issue_srcix(u, u)
            issue_dstix(u, u)
            issue_ea(u, u)
        for u in (0, 1):
            wait_srcix(u)
            issue_gather(u)
        plsc.subcore_barrier()

        def pair(t, carry):
            for u in (0, 1):
                x = 2 * t + u
                wait_gather(u)

                @pl.when(x + 2 < NCH)
                def _():
                    issue_srcix(x + 2, u)

                wait_ea(u)
                wait_dstix(u)
                compute(u)
                scatter(u)

                @pl.when(x + 2 < NCH)
                def _():
                    issue_dstix(x + 2, u)
                    issue_ea(x + 2, u)
                    wait_srcix(u)
                    issue_gather(u)
            return carry
        lax.fori_loop(0, NCH // 2, pair, 0)

        wait_gather(0)
        wait_ea(0)
        wait_dstix(0)
        compute(0)
        scatter(0)
        plsc.subcore_barrier()

        def wcopy(q, carry):
            r0 = s * WPT + q * B
            pltpu.sync_copy(agg_spm.at[pl.ds(r0, B)], eav0)
            pltpu.sync_copy(eav0, out_hbm.at[c, pl.ds(pl.multiple_of(r0, 8),
                                                      B)])
            return carry
        lax.fori_loop(0, WPT // B, wcopy, 0)

    return k(h, edge_attr, src, dst)


def _tc_layer(h, agg0, agg1, Wl, bl, gl, betal, epsl, final):
    def body(h_ref, a0_ref, a1_ref, w_ref, b_ref, g_ref, bt_ref, e_ref,
             o_ref):
        t = (1.0 + e_ref[0, 0]) * h_ref[...] + (a0_ref[...] + a1_ref[...])
        z = jnp.dot(t, w_ref[...], preferred_element_type=jnp.float32)
        z = z + b_ref[...]
        m = jnp.mean(z, axis=0, keepdims=True)
        v = jnp.mean(jnp.square(z - m), axis=0, keepdims=True)
        zn = (z - m) * lax.rsqrt(v + 1e-5) * g_ref[...] + bt_ref[...]
        o = zn + h_ref[...]
        if final:
            o = jnp.maximum(o, 0.0)
        o_ref[...] = o

    return pl.pallas_call(
        body,
        out_shape=jax.ShapeDtypeStruct((N, D), jnp.float32),
    )(h, agg0, agg1, Wl, bl.reshape(1, D), gl.reshape(1, D),
      betal.reshape(1, D), epsl.reshape(1, 1))


def kernel(x, edge_index, edge_attr, eps, W, b, gamma, beta):
    src = edge_index[0]
    dst = edge_index[1]
    h = x
    for l in range(L):
        agg = _sc_agg(h, edge_attr, src, dst)
        h = _tc_layer(h, agg[0, :N], agg[1, :N], W[l], b[l], gamma[l],
                      beta[l], eps[l], final=(l == L - 1))
    return h
